# baseline (speedup 1.0000x reference)
import jax
import jax.numpy as jnp
from jax import lax
from jax.experimental import pallas as pl
from jax.experimental.pallas import tpu as pltpu
from jax.experimental.pallas import tpu_sc as plsc

_B = 16384
_L = 200
_PAD = 1000
_TAB = 1008
_NC = 2
_NS = 16
_NW = _NC * _NS
_ROWS = _B // _NW
_LN = 16
_HALF = _ROWS // 2


def _noisy_or_body(rt_hbm, w_hbm, out_hbm, buf, tab_v, out_v, s0, s1):
    wid = lax.axis_index("s") * _NC + lax.axis_index("c")
    base = wid * _ROWS

    cp0 = pltpu.make_async_copy(
        rt_hbm.at[:, pl.ds(base, _HALF)], buf.at[:, pl.ds(0, _HALF)], s0
    )
    cp1 = pltpu.make_async_copy(
        rt_hbm.at[:, pl.ds(base + _HALF, _HALF)],
        buf.at[:, pl.ds(_HALF, _HALF)],
        s1,
    )
    cp0.start()
    cp1.start()

    pltpu.sync_copy(w_hbm, tab_v)

    lanes = lax.broadcasted_iota(jnp.int32, (_LN,), 0)

    def tbuild(j, c):
        w = tab_v[pl.ds(j * _LN, _LN)]
        p = 1.0 / (1.0 + jnp.exp(w))
        gidx = j * _LN + lanes
        tab_v[pl.ds(j * _LN, _LN)] = jnp.where(gidx == _PAD, 1.0, p)
        return c

    lax.fori_loop(0, _TAB // _LN, tbuild, 0)

    _GPB = 4

    def group_quad(h, c):
        cols = [(h * _GPB + g) * _LN for g in range(_GPB)]
        accs = [[None] * 4 for _ in range(_GPB)]
        for i in range(_L):
            k = i % 4
            for g in range(_GPB):
                ids = buf[i, pl.ds(cols[g], _LN)]
                vals = plsc.load_gather(tab_v, [ids])
                accs[g][k] = vals if accs[g][k] is None else accs[g][k] * vals
        for g in range(_GPB):
            a = accs[g]
            acc = (a[0] * a[1]) * (a[2] * a[3])
            out_v[pl.ds(cols[g], _LN)] = jnp.clip(1.0 - acc, 1e-4, 1.0 - 1e-5)
        return c

    cp0.wait()
    lax.fori_loop(0, _HALF // (_GPB * _LN), group_quad, 0)
    cp1.wait()
    lax.fori_loop(_HALF // (_GPB * _LN), _ROWS // (_GPB * _LN), group_quad, 0)

    pltpu.sync_copy(out_v, out_hbm.at[pl.ds(base, _ROWS)])


def kernel(rules, W):
    wp = jnp.concatenate(
        [W.reshape(-1).astype(jnp.float32),
         jnp.zeros((_TAB - _PAD - 1,), jnp.float32)]
    )
    f = pl.kernel(
        _noisy_or_body,
        mesh=plsc.VectorSubcoreMesh(core_axis_name="c", subcore_axis_name="s"),
        compiler_params=pltpu.CompilerParams(
            needs_layout_passes=False, use_tc_tiling_on_sc=True
        ),
        out_type=jax.ShapeDtypeStruct((_B,), jnp.float32),
        scratch_types=[
            pltpu.VMEM((_L, _ROWS), jnp.int32),
            pltpu.VMEM((_TAB,), jnp.float32),
            pltpu.VMEM((_ROWS,), jnp.float32),
            pltpu.SemaphoreType.DMA,
            pltpu.SemaphoreType.DMA,
        ],
    )
    return f(rules.astype(jnp.int32).T, wp).reshape(_B, 1)


# speedup vs baseline: 1.4642x; 1.4642x over previous
import jax
import jax.numpy as jnp
from jax import lax
from jax.experimental import pallas as pl
from jax.experimental.pallas import tpu as pltpu
from jax.experimental.pallas import tpu_sc as plsc

_B = 16384
_L = 200
_PAD = 1000
_TAB = 1008
_NC = 2
_NS = 16
_NW = _NC * _NS
_ROWS = _B // _NW
_LN = 16
_HALF = _ROWS // 2


def _noisy_or_body(rt_hbm, w_hbm, out_hbm, buf, tab_v, out_v, s0, s1):
    wid = lax.axis_index("s") * _NC + lax.axis_index("c")
    base = wid * _ROWS

    cp0 = pltpu.make_async_copy(
        rt_hbm.at[:, pl.ds(base, _HALF)], buf.at[:, pl.ds(0, _HALF)], s0
    )
    cp1 = pltpu.make_async_copy(
        rt_hbm.at[:, pl.ds(base + _HALF, _HALF)],
        buf.at[:, pl.ds(_HALF, _HALF)],
        s1,
    )
    cp0.start()
    cp1.start()

    pltpu.sync_copy(w_hbm, tab_v)

    lanes = lax.broadcasted_iota(jnp.int32, (_LN,), 0)

    def tbuild(j, c):
        w = tab_v[pl.ds(j * _LN, _LN)]
        p = 1.0 / (1.0 + jnp.exp(w))
        gidx = j * _LN + lanes
        tab_v[pl.ds(j * _LN, _LN)] = jnp.where(gidx == _PAD, 1.0, p)
        return c

    lax.fori_loop(0, _TAB // _LN, tbuild, 0)

    def group_pair(h, c):
        cols = [h * 2 * _LN, (h * 2 + 1) * _LN]
        accs = [[None] * 8, [None] * 8]
        for i in range(_L):
            k = i % 8
            for g in range(2):
                ids = buf[i, pl.ds(cols[g], _LN)]
                vals = plsc.load_gather(tab_v, [ids])
                accs[g][k] = vals if accs[g][k] is None else accs[g][k] * vals
        for g in range(2):
            a = accs[g]
            acc = ((a[0] * a[1]) * (a[2] * a[3])) * (
                (a[4] * a[5]) * (a[6] * a[7])
            )
            out_v[pl.ds(cols[g], _LN)] = jnp.clip(1.0 - acc, 1e-4, 1.0 - 1e-5)
        return c

    cp0.wait()
    lax.fori_loop(0, _HALF // (2 * _LN), group_pair, 0)
    cp1.wait()
    lax.fori_loop(_HALF // (2 * _LN), _ROWS // (2 * _LN), group_pair, 0)

    pltpu.sync_copy(out_v, out_hbm.at[pl.ds(base, _ROWS)])


def kernel(rules, W):
    wp = jnp.concatenate(
        [W.reshape(-1).astype(jnp.float32),
         jnp.zeros((_TAB - _PAD - 1,), jnp.float32)]
    )
    f = pl.kernel(
        _noisy_or_body,
        mesh=plsc.VectorSubcoreMesh(core_axis_name="c", subcore_axis_name="s"),
        compiler_params=pltpu.CompilerParams(
            needs_layout_passes=False, use_tc_tiling_on_sc=True
        ),
        out_type=jax.ShapeDtypeStruct((_B,), jnp.float32),
        scratch_types=[
            pltpu.VMEM((_L, _ROWS), jnp.int32),
            pltpu.VMEM((_TAB,), jnp.float32),
            pltpu.VMEM((_ROWS,), jnp.float32),
            pltpu.SemaphoreType.DMA,
            pltpu.SemaphoreType.DMA,
        ],
    )
    return f(rules.astype(jnp.int32).T, wp).reshape(_B, 1)


# speedup vs baseline: 1.6597x; 1.1335x over previous
import jax
import jax.numpy as jnp
from jax import lax
from jax.experimental import pallas as pl
from jax.experimental.pallas import tpu as pltpu
from jax.experimental.pallas import tpu_sc as plsc

_B = 16384
_L = 200
_PAD = 1000
_TAB = 1008
_NC = 2
_NS = 16
_NW = _NC * _NS
_ROWS = _B // _NW
_LN = 16
_Q = _ROWS // 4


def _noisy_or_body(rt_hbm, w_hbm, out_hbm, buf, tab_v, out_v, sw, s0, s1, s2, s3):
    wid = lax.axis_index("s") * _NC + lax.axis_index("c")
    base = wid * _ROWS

    cw = pltpu.make_async_copy(w_hbm, tab_v, sw)
    cw.start()
    qsems = [s0, s1, s2, s3]
    qcopies = [
        pltpu.make_async_copy(
            rt_hbm.at[:, pl.ds(base + q * _Q, _Q)],
            buf.at[:, pl.ds(q * _Q, _Q)],
            qsems[q],
        )
        for q in range(4)
    ]
    for cp in qcopies:
        cp.start()

    lanes = lax.broadcasted_iota(jnp.int32, (_LN,), 0)

    cw.wait()

    def tbuild(j, c):
        w = tab_v[pl.ds(j * _LN, _LN)]
        p = 1.0 / (1.0 + jnp.exp(w))
        gidx = j * _LN + lanes
        tab_v[pl.ds(j * _LN, _LN)] = jnp.where(gidx == _PAD, 1.0, p)
        return c

    lax.fori_loop(0, _TAB // _LN, tbuild, 0)

    def group_pair(h, c):
        for q in range(4):
            @pl.when(h == q * (_Q // (2 * _LN)))
            def _(q=q):
                qcopies[q].wait()

        cols = [h * 2 * _LN, (h * 2 + 1) * _LN]
        accs = [[None] * 8, [None] * 8]
        for i in range(_L):
            k = i % 8
            for g in range(2):
                ids = buf[i, pl.ds(cols[g], _LN)]
                vals = plsc.load_gather(tab_v, [ids])
                accs[g][k] = vals if accs[g][k] is None else accs[g][k] * vals
        for g in range(2):
            a = accs[g]
            acc = ((a[0] * a[1]) * (a[2] * a[3])) * (
                (a[4] * a[5]) * (a[6] * a[7])
            )
            out_v[pl.ds(cols[g], _LN)] = jnp.clip(1.0 - acc, 1e-4, 1.0 - 1e-5)
        return c

    lax.fori_loop(0, _ROWS // (2 * _LN), group_pair, 0)

    pltpu.sync_copy(out_v, out_hbm.at[pl.ds(base, _ROWS)])


def kernel(rules, W):
    f = pl.kernel(
        _noisy_or_body,
        mesh=plsc.VectorSubcoreMesh(core_axis_name="c", subcore_axis_name="s"),
        compiler_params=pltpu.CompilerParams(
            needs_layout_passes=False, use_tc_tiling_on_sc=True
        ),
        out_type=jax.ShapeDtypeStruct((_B,), jnp.float32),
        scratch_types=[
            pltpu.VMEM((_L, _ROWS), jnp.int32),
            pltpu.VMEM((_TAB,), jnp.float32),
            pltpu.VMEM((_ROWS,), jnp.float32),
            pltpu.SemaphoreType.DMA,
            pltpu.SemaphoreType.DMA,
            pltpu.SemaphoreType.DMA,
            pltpu.SemaphoreType.DMA,
            pltpu.SemaphoreType.DMA,
        ],
    )
    wp = jnp.concatenate(
        [W.reshape(-1).astype(jnp.float32),
         jnp.zeros((_TAB - _PAD - 1,), jnp.float32)]
    )
    return f(rules.astype(jnp.int32).T, wp).reshape(_B, 1)
